# baseline (speedup 1.0000x reference)
import functools

import jax
import jax.numpy as jnp
from jax import lax
from jax.experimental import pallas as pl
from jax.experimental.pallas import tpu as pltpu
from jax.experimental.pallas import tpu_sc as plsc

NUM_NODES = 100000
MEM_DIM = 256
HID = 128
B = 16384

NC = 2
NS = 16
NW = NC * NS
CHUNK = 128

NXC = 4
BC = B // NXC
BLK = 2048


def _sc_gather(memory, idx, nrows):
    rows_per_w = nrows // NW
    nchunk = rows_per_w // CHUNK
    idx3 = idx.reshape(NW, nchunk, CHUNK)
    mesh = plsc.VectorSubcoreMesh(
        core_axis_name="c", subcore_axis_name="s", num_cores=NC, num_subcores=NS
    )

    @functools.partial(
        pl.kernel,
        out_type=jax.ShapeDtypeStruct((nrows, MEM_DIM), jnp.float32),
        mesh=mesh,
        scratch_types=[
            pltpu.VMEM((nchunk, CHUNK), jnp.int32),
            pltpu.VMEM((CHUNK, MEM_DIM), jnp.float32),
            pltpu.VMEM((CHUNK, MEM_DIM), jnp.float32),
            pltpu.SemaphoreType.DMA,
            pltpu.SemaphoreType.DMA,
            pltpu.SemaphoreType.DMA,
            pltpu.SemaphoreType.DMA,
        ],
    )
    def gather_kernel(mem_hbm, idx_hbm, out_hbm, idx_v, rows0, rows1,
                      gsem0, gsem1, wsem0, wsem1):
        wid = lax.axis_index("s") * NC + lax.axis_index("c")
        pltpu.sync_copy(idx_hbm.at[wid], idx_v)
        base = wid * rows_per_w
        bufs = (rows0, rows1)
        gsems = (gsem0, gsem1)
        wsems = (wsem0, wsem1)
        gd = [None] * nchunk
        wd = [None] * nchunk
        gd[0] = pltpu.async_copy(mem_hbm.at[idx_v.at[0]], bufs[0], gsems[0])
        for c in range(nchunk):
            p = c % 2
            gd[c].wait()
            wd[c] = pltpu.async_copy(
                bufs[p], out_hbm.at[pl.ds(base + c * CHUNK, CHUNK)], wsems[p])
            if c + 1 < nchunk:
                if c >= 1:
                    wd[c - 1].wait()
                gd[c + 1] = pltpu.async_copy(
                    mem_hbm.at[idx_v.at[c + 1]], bufs[(c + 1) % 2],
                    gsems[(c + 1) % 2])
        if nchunk >= 2:
            wd[nchunk - 2].wait()
        wd[nchunk - 1].wait()

    return gather_kernel(memory, idx3)


def _tc_body(src_ref, dst_ref, t_ref,
             w1s_ref, w1d_ref, w1t_ref, b1_ref, w2_ref, b2_ref,
             wih_ref, bih_ref, whh_ref, bhh_ref,
             p1a_ref, p1b_ref, pb1_ref, p2_ref, pb2_ref,
             score_in, ns_in, nd_in,
             score_ref, ns_ref, nd_ref):
    del score_in, ns_in, nd_in
    s = src_ref[...]
    d = dst_ref[...]
    t = t_ref[...]

    bf = jnp.bfloat16
    f32 = jnp.float32
    dot = lambda a, b: jax.lax.dot(a.astype(bf), b.astype(bf),
                                   preferred_element_type=f32)
    h = dot(s, w1s_ref[...]) + dot(d, w1d_ref[...]) + t * w1t_ref[...] + b1_ref[...]
    h = jnp.maximum(h, 0.0)
    msg = dot(h, w2_ref[...]) + b2_ref[...]

    gi = dot(msg, wih_ref[...]) + bih_ref[...]
    gh_s = dot(s, whh_ref[...]) + bhh_ref[...]
    gh_d = dot(d, whh_ref[...]) + bhh_ref[...]

    def sigmoid(x):
        return 0.5 * jnp.tanh(0.5 * x) + 0.5

    def gru(gi, gh, hprev):
        i_r = gi[:, :MEM_DIM]
        i_z = gi[:, MEM_DIM:2 * MEM_DIM]
        i_n = gi[:, 2 * MEM_DIM:]
        h_r = gh[:, :MEM_DIM]
        h_z = gh[:, MEM_DIM:2 * MEM_DIM]
        h_n = gh[:, 2 * MEM_DIM:]
        r = sigmoid(i_r + h_r)
        z = sigmoid(i_z + h_z)
        n = jnp.tanh(i_n + r * h_n)
        return (1.0 - z) * n + z * hprev

    new_s = gru(gi, gh_s, s)
    new_d = gru(gi, gh_d, d)
    ns_ref[...] = new_s
    nd_ref[...] = new_d

    ph = dot(new_s, p1a_ref[...]) + dot(new_d, p1b_ref[...]) + pb1_ref[...]
    ph = jnp.maximum(ph, 0.0)
    score_row = (
        jax.lax.dot_general(p2_ref[...].astype(bf), ph.astype(bf),
                            (((1,), (1,)), ((), ())),
                            preferred_element_type=f32)
        + pb2_ref[...])
    score_ref[...] = score_row.reshape(score_ref.shape)


def _make_tc_call(k, aliased):
    nblk = BC // BLK
    off = k * nblk
    crow = lambda i: (i, 0)
    cdst = lambda i: (i + nblk, 0)
    grow = lambda i: (i + off, 0)
    const = lambda i: (0, 0)
    any_spec = pl.BlockSpec(memory_space=pl.ANY)
    in_specs = [
        pl.BlockSpec((BLK, MEM_DIM), crow),
        pl.BlockSpec((BLK, MEM_DIM), cdst),
        pl.BlockSpec((BLK, 1), grow),
        pl.BlockSpec((MEM_DIM, HID), const),
        pl.BlockSpec((MEM_DIM, HID), const),
        pl.BlockSpec((1, HID), const),
        pl.BlockSpec((1, HID), const),
        pl.BlockSpec((HID, MEM_DIM), const),
        pl.BlockSpec((1, MEM_DIM), const),
        pl.BlockSpec((MEM_DIM, 3 * MEM_DIM), const),
        pl.BlockSpec((1, 3 * MEM_DIM), const),
        pl.BlockSpec((MEM_DIM, 3 * MEM_DIM), const),
        pl.BlockSpec((1, 3 * MEM_DIM), const),
        pl.BlockSpec((MEM_DIM, HID), const),
        pl.BlockSpec((MEM_DIM, HID), const),
        pl.BlockSpec((1, HID), const),
        pl.BlockSpec((1, HID), const),
        pl.BlockSpec((1, 1), const),
    ]
    if aliased:
        in_specs += [any_spec, any_spec, any_spec]
    out_specs = [
        pl.BlockSpec((1, 1, BLK), lambda i: (i + off, 0, 0)),
        pl.BlockSpec((BLK, MEM_DIM), grow),
        pl.BlockSpec((BLK, MEM_DIM), grow),
    ]
    out_shape = [
        jax.ShapeDtypeStruct((B // BLK, 1, BLK), jnp.float32),
        jax.ShapeDtypeStruct((B, MEM_DIM), jnp.float32),
        jax.ShapeDtypeStruct((B, MEM_DIM), jnp.float32),
    ]
    if aliased:
        body = _tc_body
    else:
        def body(*args):
            return _tc_body(*args[:18], None, None, None, *args[18:])
    return pl.pallas_call(
        body,
        grid=(nblk,),
        in_specs=in_specs,
        out_specs=out_specs,
        out_shape=out_shape,
        input_output_aliases={18: 0, 19: 1, 20: 2} if aliased else {},
        compiler_params=pltpu.CompilerParams(
            dimension_semantics=("arbitrary",),
        ),
    )


def kernel(src, dst, time, memory, mmlp_w1, mmlp_b1, mmlp_w2, mmlp_b2,
           gru_w_ih, gru_w_hh, gru_b_ih, gru_b_hh,
           pred_w1, pred_b1, pred_w2, pred_b2):
    src = src.astype(jnp.int32)
    dst = dst.astype(jnp.int32)
    t = time.reshape(B, 1)
    w1s = mmlp_w1[:, :MEM_DIM].T
    w1d = mmlp_w1[:, MEM_DIM:2 * MEM_DIM].T
    w1t = mmlp_w1[:, 2 * MEM_DIM].reshape(1, HID)
    b1 = mmlp_b1.reshape(1, HID)
    w2 = mmlp_w2.T
    b2 = mmlp_b2.reshape(1, MEM_DIM)
    wih = gru_w_ih.T
    bih = gru_b_ih.reshape(1, 3 * MEM_DIM)
    whh = gru_w_hh.T
    bhh = gru_b_hh.reshape(1, 3 * MEM_DIM)
    p1a = pred_w1[:, :MEM_DIM].T
    p1b = pred_w1[:, MEM_DIM:].T
    pb1 = pred_b1.reshape(1, HID)
    p2 = pred_w2.reshape(1, HID)
    pb2 = pred_b2.reshape(1, 1)

    staged = []
    for k in range(NXC):
        idx_k = jnp.concatenate(
            [lax.dynamic_slice_in_dim(src, k * BC, BC),
             lax.dynamic_slice_in_dim(dst, k * BC, BC)])
        staged.append(_sc_gather(memory, idx_k, 2 * BC))

    score = ns = nd = None
    for k in range(NXC):
        weights = (w1s, w1d, w1t, b1, w2, b2, wih, bih, whh, bhh,
                   p1a, p1b, pb1, p2, pb2)
        if k == 0:
            score, ns, nd = _make_tc_call(0, aliased=False)(
                staged[0], staged[0], t, *weights)
        else:
            score, ns, nd = _make_tc_call(k, aliased=True)(
                staged[k], staged[k], t, *weights, score, ns, nd)
    return (score.reshape(B, 1), ns, nd)


# speedup vs baseline: 1.1374x; 1.1374x over previous
import functools

import jax
import jax.numpy as jnp
from jax import lax
from jax.experimental import pallas as pl
from jax.experimental.pallas import tpu as pltpu
from jax.experimental.pallas import tpu_sc as plsc

NUM_NODES = 100000
MEM_DIM = 256
HID = 128
B = 16384

NC = 2
NS = 16
NW = NC * NS
CHUNK = 128

NXC = 2
BC = B // NXC
BLK = 2048


def _sc_gather(memory, idx, nrows):
    rows_per_w = nrows // NW
    nchunk = rows_per_w // CHUNK
    idx3 = idx.reshape(NW, nchunk, CHUNK)
    mesh = plsc.VectorSubcoreMesh(
        core_axis_name="c", subcore_axis_name="s", num_cores=NC, num_subcores=NS
    )

    @functools.partial(
        pl.kernel,
        out_type=jax.ShapeDtypeStruct((nrows, MEM_DIM), jnp.float32),
        mesh=mesh,
        scratch_types=[
            pltpu.VMEM((nchunk, CHUNK), jnp.int32),
            pltpu.VMEM((CHUNK, MEM_DIM), jnp.float32),
            pltpu.VMEM((CHUNK, MEM_DIM), jnp.float32),
            pltpu.SemaphoreType.DMA,
            pltpu.SemaphoreType.DMA,
            pltpu.SemaphoreType.DMA,
            pltpu.SemaphoreType.DMA,
        ],
    )
    def gather_kernel(mem_hbm, idx_hbm, out_hbm, idx_v, rows0, rows1,
                      gsem0, gsem1, wsem0, wsem1):
        wid = lax.axis_index("s") * NC + lax.axis_index("c")
        pltpu.sync_copy(idx_hbm.at[wid], idx_v)
        base = wid * rows_per_w
        bufs = (rows0, rows1)
        gsems = (gsem0, gsem1)
        wsems = (wsem0, wsem1)
        gd = [None] * nchunk
        wd = [None] * nchunk
        gd[0] = pltpu.async_copy(mem_hbm.at[idx_v.at[0]], bufs[0], gsems[0])
        for c in range(nchunk):
            p = c % 2
            gd[c].wait()
            wd[c] = pltpu.async_copy(
                bufs[p], out_hbm.at[pl.ds(base + c * CHUNK, CHUNK)], wsems[p])
            if c + 1 < nchunk:
                if c >= 1:
                    wd[c - 1].wait()
                gd[c + 1] = pltpu.async_copy(
                    mem_hbm.at[idx_v.at[c + 1]], bufs[(c + 1) % 2],
                    gsems[(c + 1) % 2])
        if nchunk >= 2:
            wd[nchunk - 2].wait()
        wd[nchunk - 1].wait()

    return gather_kernel(memory, idx3)


def _tc_body(src_ref, dst_ref, t_ref,
             w1s_ref, w1d_ref, w1t_ref, b1_ref, w2_ref, b2_ref,
             wih_ref, bih_ref, whh_ref, bhh_ref,
             p1a_ref, p1b_ref, pb1_ref, p2_ref, pb2_ref,
             score_in, ns_in, nd_in,
             score_ref, ns_ref, nd_ref):
    del score_in, ns_in, nd_in
    s = src_ref[...]
    d = dst_ref[...]
    t = t_ref[...]

    bf = jnp.bfloat16
    f32 = jnp.float32
    dot = lambda a, b: jax.lax.dot(a.astype(bf), b.astype(bf),
                                   preferred_element_type=f32)
    h = dot(s, w1s_ref[...]) + dot(d, w1d_ref[...]) + t * w1t_ref[...] + b1_ref[...]
    h = jnp.maximum(h, 0.0)
    msg = dot(h, w2_ref[...]) + b2_ref[...]

    gi = dot(msg, wih_ref[...]) + bih_ref[...]
    gh_s = dot(s, whh_ref[...]) + bhh_ref[...]
    gh_d = dot(d, whh_ref[...]) + bhh_ref[...]

    def sigmoid(x):
        return 0.5 * jnp.tanh(0.5 * x) + 0.5

    def gru(gi, gh, hprev):
        i_r = gi[:, :MEM_DIM]
        i_z = gi[:, MEM_DIM:2 * MEM_DIM]
        i_n = gi[:, 2 * MEM_DIM:]
        h_r = gh[:, :MEM_DIM]
        h_z = gh[:, MEM_DIM:2 * MEM_DIM]
        h_n = gh[:, 2 * MEM_DIM:]
        r = sigmoid(i_r + h_r)
        z = sigmoid(i_z + h_z)
        n = jnp.tanh(i_n + r * h_n)
        return (1.0 - z) * n + z * hprev

    new_s = gru(gi, gh_s, s)
    new_d = gru(gi, gh_d, d)
    ns_ref[...] = new_s
    nd_ref[...] = new_d

    ph = dot(new_s, p1a_ref[...]) + dot(new_d, p1b_ref[...]) + pb1_ref[...]
    ph = jnp.maximum(ph, 0.0)
    score_row = (
        jax.lax.dot_general(p2_ref[...].astype(bf), ph.astype(bf),
                            (((1,), (1,)), ((), ())),
                            preferred_element_type=f32)
        + pb2_ref[...])
    score_ref[...] = score_row.reshape(score_ref.shape)


def _make_tc_call(k, aliased):
    nblk = BC // BLK
    off = k * nblk
    crow = lambda i: (i, 0)
    cdst = lambda i: (i + nblk, 0)
    grow = lambda i: (i + off, 0)
    const = lambda i: (0, 0)
    any_spec = pl.BlockSpec(memory_space=pl.ANY)
    in_specs = [
        pl.BlockSpec((BLK, MEM_DIM), crow),
        pl.BlockSpec((BLK, MEM_DIM), cdst),
        pl.BlockSpec((BLK, 1), grow),
        pl.BlockSpec((MEM_DIM, HID), const),
        pl.BlockSpec((MEM_DIM, HID), const),
        pl.BlockSpec((1, HID), const),
        pl.BlockSpec((1, HID), const),
        pl.BlockSpec((HID, MEM_DIM), const),
        pl.BlockSpec((1, MEM_DIM), const),
        pl.BlockSpec((MEM_DIM, 3 * MEM_DIM), const),
        pl.BlockSpec((1, 3 * MEM_DIM), const),
        pl.BlockSpec((MEM_DIM, 3 * MEM_DIM), const),
        pl.BlockSpec((1, 3 * MEM_DIM), const),
        pl.BlockSpec((MEM_DIM, HID), const),
        pl.BlockSpec((MEM_DIM, HID), const),
        pl.BlockSpec((1, HID), const),
        pl.BlockSpec((1, HID), const),
        pl.BlockSpec((1, 1), const),
    ]
    if aliased:
        in_specs += [any_spec, any_spec, any_spec]
    out_specs = [
        pl.BlockSpec((1, 1, BLK), lambda i: (i + off, 0, 0)),
        pl.BlockSpec((BLK, MEM_DIM), grow),
        pl.BlockSpec((BLK, MEM_DIM), grow),
    ]
    out_shape = [
        jax.ShapeDtypeStruct((B // BLK, 1, BLK), jnp.float32),
        jax.ShapeDtypeStruct((B, MEM_DIM), jnp.float32),
        jax.ShapeDtypeStruct((B, MEM_DIM), jnp.float32),
    ]
    if aliased:
        body = _tc_body
    else:
        def body(*args):
            return _tc_body(*args[:18], None, None, None, *args[18:])
    return pl.pallas_call(
        body,
        grid=(nblk,),
        in_specs=in_specs,
        out_specs=out_specs,
        out_shape=out_shape,
        input_output_aliases={18: 0, 19: 1, 20: 2} if aliased else {},
        compiler_params=pltpu.CompilerParams(
            dimension_semantics=("arbitrary",),
        ),
    )


def kernel(src, dst, time, memory, mmlp_w1, mmlp_b1, mmlp_w2, mmlp_b2,
           gru_w_ih, gru_w_hh, gru_b_ih, gru_b_hh,
           pred_w1, pred_b1, pred_w2, pred_b2):
    src = src.astype(jnp.int32)
    dst = dst.astype(jnp.int32)
    t = time.reshape(B, 1)
    w1s = mmlp_w1[:, :MEM_DIM].T
    w1d = mmlp_w1[:, MEM_DIM:2 * MEM_DIM].T
    w1t = mmlp_w1[:, 2 * MEM_DIM].reshape(1, HID)
    b1 = mmlp_b1.reshape(1, HID)
    w2 = mmlp_w2.T
    b2 = mmlp_b2.reshape(1, MEM_DIM)
    wih = gru_w_ih.T
    bih = gru_b_ih.reshape(1, 3 * MEM_DIM)
    whh = gru_w_hh.T
    bhh = gru_b_hh.reshape(1, 3 * MEM_DIM)
    p1a = pred_w1[:, :MEM_DIM].T
    p1b = pred_w1[:, MEM_DIM:].T
    pb1 = pred_b1.reshape(1, HID)
    p2 = pred_w2.reshape(1, HID)
    pb2 = pred_b2.reshape(1, 1)

    staged = []
    for k in range(NXC):
        idx_k = jnp.concatenate(
            [lax.dynamic_slice_in_dim(src, k * BC, BC),
             lax.dynamic_slice_in_dim(dst, k * BC, BC)])
        staged.append(_sc_gather(memory, idx_k, 2 * BC))

    score = ns = nd = None
    for k in range(NXC):
        weights = (w1s, w1d, w1t, b1, w2, b2, wih, bih, whh, bhh,
                   p1a, p1b, pb1, p2, pb2)
        if k == 0:
            score, ns, nd = _make_tc_call(0, aliased=False)(
                staged[0], staged[0], t, *weights)
        else:
            score, ns, nd = _make_tc_call(k, aliased=True)(
                staged[k], staged[k], t, *weights, score, ns, nd)
    return (score.reshape(B, 1), ns, nd)


# speedup vs baseline: 1.1577x; 1.0179x over previous
import functools

import jax
import jax.numpy as jnp
from jax import lax
from jax.experimental import pallas as pl
from jax.experimental.pallas import tpu as pltpu
from jax.experimental.pallas import tpu_sc as plsc

NUM_NODES = 100000
MEM_DIM = 256
HID = 128
B = 16384

NC = 2
NS = 16
NW = NC * NS
CHUNK = 128

CHUNK_ROWS = (6144, 10240)
BLK = 2048


def _sc_gather(memory, idx, nrows):
    rows_per_w = nrows // NW
    nchunk = rows_per_w // CHUNK
    idx3 = idx.reshape(NW, nchunk, CHUNK)
    mesh = plsc.VectorSubcoreMesh(
        core_axis_name="c", subcore_axis_name="s", num_cores=NC, num_subcores=NS
    )

    @functools.partial(
        pl.kernel,
        out_type=jax.ShapeDtypeStruct((nrows, MEM_DIM), jnp.float32),
        mesh=mesh,
        scratch_types=[
            pltpu.VMEM((nchunk, CHUNK), jnp.int32),
            pltpu.VMEM((CHUNK, MEM_DIM), jnp.float32),
            pltpu.VMEM((CHUNK, MEM_DIM), jnp.float32),
            pltpu.SemaphoreType.DMA,
            pltpu.SemaphoreType.DMA,
            pltpu.SemaphoreType.DMA,
            pltpu.SemaphoreType.DMA,
        ],
    )
    def gather_kernel(mem_hbm, idx_hbm, out_hbm, idx_v, rows0, rows1,
                      gsem0, gsem1, wsem0, wsem1):
        wid = lax.axis_index("s") * NC + lax.axis_index("c")
        pltpu.sync_copy(idx_hbm.at[wid], idx_v)
        base = wid * rows_per_w
        bufs = (rows0, rows1)
        gsems = (gsem0, gsem1)
        wsems = (wsem0, wsem1)
        gd = [None] * nchunk
        wd = [None] * nchunk
        gd[0] = pltpu.async_copy(mem_hbm.at[idx_v.at[0]], bufs[0], gsems[0])
        for c in range(nchunk):
            p = c % 2
            gd[c].wait()
            wd[c] = pltpu.async_copy(
                bufs[p], out_hbm.at[pl.ds(base + c * CHUNK, CHUNK)], wsems[p])
            if c + 1 < nchunk:
                if c >= 1:
                    wd[c - 1].wait()
                gd[c + 1] = pltpu.async_copy(
                    mem_hbm.at[idx_v.at[c + 1]], bufs[(c + 1) % 2],
                    gsems[(c + 1) % 2])
        if nchunk >= 2:
            wd[nchunk - 2].wait()
        wd[nchunk - 1].wait()

    return gather_kernel(memory, idx3)


def _tc_body(src_ref, dst_ref, t_ref,
             w1s_ref, w1d_ref, w1t_ref, b1_ref, w2_ref, b2_ref,
             wih_ref, bih_ref, whh_ref, bhh_ref,
             p1a_ref, p1b_ref, pb1_ref, p2_ref, pb2_ref,
             score_in, ns_in, nd_in,
             score_ref, ns_ref, nd_ref):
    del score_in, ns_in, nd_in
    s = src_ref[...]
    d = dst_ref[...]
    t = t_ref[...]

    bf = jnp.bfloat16
    f32 = jnp.float32
    dot = lambda a, b: jax.lax.dot(a.astype(bf), b.astype(bf),
                                   preferred_element_type=f32)
    h = dot(s, w1s_ref[...]) + dot(d, w1d_ref[...]) + t * w1t_ref[...] + b1_ref[...]
    h = jnp.maximum(h, 0.0)
    msg = dot(h, w2_ref[...]) + b2_ref[...]

    gi = dot(msg, wih_ref[...]) + bih_ref[...]
    gh_s = dot(s, whh_ref[...]) + bhh_ref[...]
    gh_d = dot(d, whh_ref[...]) + bhh_ref[...]

    def gru(gi, gh, hprev):
        i_r = gi[:, :MEM_DIM]
        i_z = gi[:, MEM_DIM:2 * MEM_DIM]
        i_n = gi[:, 2 * MEM_DIM:]
        h_r = gh[:, :MEM_DIM]
        h_z = gh[:, MEM_DIM:2 * MEM_DIM]
        h_n = gh[:, 2 * MEM_DIM:]
        r = 0.5 * jnp.tanh(i_r + h_r) + 0.5
        z = 0.5 * jnp.tanh(i_z + h_z) + 0.5
        n = jnp.tanh(i_n + r * h_n)
        return n + z * (hprev - n)

    new_s = gru(gi, gh_s, s)
    new_d = gru(gi, gh_d, d)
    ns_ref[...] = new_s
    nd_ref[...] = new_d

    ph = dot(new_s, p1a_ref[...]) + dot(new_d, p1b_ref[...]) + pb1_ref[...]
    ph = jnp.maximum(ph, 0.0)
    score_row = (
        jax.lax.dot_general(p2_ref[...].astype(bf), ph.astype(bf),
                            (((1,), (1,)), ((), ())),
                            preferred_element_type=f32)
        + pb2_ref[...])
    score_ref[...] = score_row.reshape(score_ref.shape)


def _make_tc_call(row_off, bc, aliased):
    nblk = bc // BLK
    off = row_off // BLK
    crow = lambda i: (i, 0)
    cdst = lambda i: (i + nblk, 0)
    grow = lambda i: (i + off, 0)
    const = lambda i: (0, 0)
    any_spec = pl.BlockSpec(memory_space=pl.ANY)
    in_specs = [
        pl.BlockSpec((BLK, MEM_DIM), crow),
        pl.BlockSpec((BLK, MEM_DIM), cdst),
        pl.BlockSpec((BLK, 1), grow),
        pl.BlockSpec((MEM_DIM, HID), const),
        pl.BlockSpec((MEM_DIM, HID), const),
        pl.BlockSpec((1, HID), const),
        pl.BlockSpec((1, HID), const),
        pl.BlockSpec((HID, MEM_DIM), const),
        pl.BlockSpec((1, MEM_DIM), const),
        pl.BlockSpec((MEM_DIM, 3 * MEM_DIM), const),
        pl.BlockSpec((1, 3 * MEM_DIM), const),
        pl.BlockSpec((MEM_DIM, 3 * MEM_DIM), const),
        pl.BlockSpec((1, 3 * MEM_DIM), const),
        pl.BlockSpec((MEM_DIM, HID), const),
        pl.BlockSpec((MEM_DIM, HID), const),
        pl.BlockSpec((1, HID), const),
        pl.BlockSpec((1, HID), const),
        pl.BlockSpec((1, 1), const),
    ]
    if aliased:
        in_specs += [any_spec, any_spec, any_spec]
    out_specs = [
        pl.BlockSpec((1, 1, BLK), lambda i: (i + off, 0, 0)),
        pl.BlockSpec((BLK, MEM_DIM), grow),
        pl.BlockSpec((BLK, MEM_DIM), grow),
    ]
    out_shape = [
        jax.ShapeDtypeStruct((B // BLK, 1, BLK), jnp.float32),
        jax.ShapeDtypeStruct((B, MEM_DIM), jnp.float32),
        jax.ShapeDtypeStruct((B, MEM_DIM), jnp.float32),
    ]
    if aliased:
        body = _tc_body
    else:
        def body(*args):
            return _tc_body(*args[:18], None, None, None, *args[18:])
    return pl.pallas_call(
        body,
        grid=(nblk,),
        in_specs=in_specs,
        out_specs=out_specs,
        out_shape=out_shape,
        input_output_aliases={18: 0, 19: 1, 20: 2} if aliased else {},
        compiler_params=pltpu.CompilerParams(
            dimension_semantics=("arbitrary",),
        ),
    )


def kernel(src, dst, time, memory, mmlp_w1, mmlp_b1, mmlp_w2, mmlp_b2,
           gru_w_ih, gru_w_hh, gru_b_ih, gru_b_hh,
           pred_w1, pred_b1, pred_w2, pred_b2):
    src = src.astype(jnp.int32)
    dst = dst.astype(jnp.int32)
    t = time.reshape(B, 1)
    w1s = mmlp_w1[:, :MEM_DIM].T
    w1d = mmlp_w1[:, MEM_DIM:2 * MEM_DIM].T
    w1t = mmlp_w1[:, 2 * MEM_DIM].reshape(1, HID)
    b1 = mmlp_b1.reshape(1, HID)
    w2 = mmlp_w2.T
    b2 = mmlp_b2.reshape(1, MEM_DIM)
    gate_scale = jnp.concatenate(
        [jnp.full((1, 2 * MEM_DIM), 0.5, jnp.float32),
         jnp.ones((1, MEM_DIM), jnp.float32)], axis=1)
    wih = gru_w_ih.T * gate_scale
    bih = gru_b_ih.reshape(1, 3 * MEM_DIM) * gate_scale
    whh = gru_w_hh.T * gate_scale
    bhh = gru_b_hh.reshape(1, 3 * MEM_DIM) * gate_scale
    p1a = pred_w1[:, :MEM_DIM].T
    p1b = pred_w1[:, MEM_DIM:].T
    pb1 = pred_b1.reshape(1, HID)
    p2 = pred_w2.reshape(1, HID)
    pb2 = pred_b2.reshape(1, 1)

    offs = [0]
    for bc in CHUNK_ROWS[:-1]:
        offs.append(offs[-1] + bc)

    staged = []
    for off, bc in zip(offs, CHUNK_ROWS):
        idx_k = jnp.concatenate(
            [lax.dynamic_slice_in_dim(src, off, bc),
             lax.dynamic_slice_in_dim(dst, off, bc)])
        staged.append(_sc_gather(memory, idx_k, 2 * bc))

    weights = (w1s, w1d, w1t, b1, w2, b2, wih, bih, whh, bhh,
               p1a, p1b, pb1, p2, pb2)
    score = ns = nd = None
    for k, (off, bc) in enumerate(zip(offs, CHUNK_ROWS)):
        if k == 0:
            score, ns, nd = _make_tc_call(off, bc, aliased=False)(
                staged[k], staged[k], t, *weights)
        else:
            score, ns, nd = _make_tc_call(off, bc, aliased=True)(
                staged[k], staged[k], t, *weights, score, ns, nd)
    return (score.reshape(B, 1), ns, nd)


# speedup vs baseline: 1.2365x; 1.0680x over previous
import functools

import jax
import jax.numpy as jnp
from jax import lax
from jax.experimental import pallas as pl
from jax.experimental.pallas import tpu as pltpu
from jax.experimental.pallas import tpu_sc as plsc

NUM_NODES = 100000
MEM_DIM = 256
HID = 128
B = 16384

NC = 2
NS = 16
NW = NC * NS
CHUNK = 128

CHUNK_ROWS = (6144, 10240)
BLK = 2048


def _sc_gather(memory, idx, nrows):
    rows_per_w = nrows // NW
    nchunk = rows_per_w // CHUNK
    idx3 = idx.reshape(NW, nchunk, CHUNK)
    mesh = plsc.VectorSubcoreMesh(
        core_axis_name="c", subcore_axis_name="s", num_cores=NC, num_subcores=NS
    )

    @functools.partial(
        pl.kernel,
        out_type=jax.ShapeDtypeStruct((nrows, MEM_DIM), jnp.float32),
        mesh=mesh,
        scratch_types=[
            pltpu.VMEM((nchunk, CHUNK), jnp.int32),
            pltpu.VMEM((CHUNK, MEM_DIM), jnp.float32),
            pltpu.VMEM((CHUNK, MEM_DIM), jnp.float32),
            pltpu.SemaphoreType.DMA,
            pltpu.SemaphoreType.DMA,
            pltpu.SemaphoreType.DMA,
            pltpu.SemaphoreType.DMA,
        ],
    )
    def gather_kernel(mem_hbm, idx_hbm, out_hbm, idx_v, rows0, rows1,
                      gsem0, gsem1, wsem0, wsem1):
        wid = lax.axis_index("s") * NC + lax.axis_index("c")
        pltpu.sync_copy(idx_hbm.at[wid], idx_v)
        base = wid * rows_per_w
        bufs = (rows0, rows1)
        gsems = (gsem0, gsem1)
        wsems = (wsem0, wsem1)
        gd = [None] * nchunk
        wd = [None] * nchunk
        gd[0] = pltpu.async_copy(mem_hbm.at[idx_v.at[0]], bufs[0], gsems[0])
        for c in range(nchunk):
            p = c % 2
            gd[c].wait()
            wd[c] = pltpu.async_copy(
                bufs[p], out_hbm.at[pl.ds(base + c * CHUNK, CHUNK)], wsems[p])
            if c + 1 < nchunk:
                if c >= 1:
                    wd[c - 1].wait()
                gd[c + 1] = pltpu.async_copy(
                    mem_hbm.at[idx_v.at[c + 1]], bufs[(c + 1) % 2],
                    gsems[(c + 1) % 2])
        if nchunk >= 2:
            wd[nchunk - 2].wait()
        wd[nchunk - 1].wait()

    return gather_kernel(memory, idx3)


def _tc_body(src_ref, dst_ref, t_ref,
             w1s_ref, w1d_ref, w1t_ref, b1_ref, w2_ref, b2_ref,
             wih_ref, bih_ref, whh_ref, bhh_ref,
             p1a_ref, p1b_ref, pb1_ref, p2_ref, pb2_ref,
             score_in, ns_in, nd_in,
             score_ref, ns_ref, nd_ref):
    del score_in, ns_in, nd_in
    s = src_ref[...]
    d = dst_ref[...]
    t = t_ref[...]

    bf = jnp.bfloat16
    f32 = jnp.float32
    dot = lambda a, b: jax.lax.dot(a.astype(bf), b.astype(bf),
                                   preferred_element_type=f32)
    h = dot(s, w1s_ref[...]) + dot(d, w1d_ref[...]) + t * w1t_ref[...] + b1_ref[...]
    h = jnp.maximum(h, 0.0)
    msg = dot(h, w2_ref[...]) + b2_ref[...]

    gi = (dot(msg, wih_ref[...]) + bih_ref[...]).astype(bf)
    gh_s = (dot(s, whh_ref[...]) + bhh_ref[...]).astype(bf)
    gh_d = (dot(d, whh_ref[...]) + bhh_ref[...]).astype(bf)

    half = jnp.bfloat16(0.5)

    def gru(gi, gh, hprev):
        i_r = gi[:, :MEM_DIM]
        i_z = gi[:, MEM_DIM:2 * MEM_DIM]
        i_n = gi[:, 2 * MEM_DIM:]
        h_r = gh[:, :MEM_DIM]
        h_z = gh[:, MEM_DIM:2 * MEM_DIM]
        h_n = gh[:, 2 * MEM_DIM:]
        r = half * jnp.tanh(i_r + h_r) + half
        z = half * jnp.tanh(i_z + h_z) + half
        n = jnp.tanh(i_n + r * h_n)
        return n + z * (hprev.astype(bf) - n)

    new_s = gru(gi, gh_s, s).astype(f32)
    new_d = gru(gi, gh_d, d).astype(f32)
    ns_ref[...] = new_s
    nd_ref[...] = new_d

    ph = dot(new_s, p1a_ref[...]) + dot(new_d, p1b_ref[...]) + pb1_ref[...]
    ph = jnp.maximum(ph, 0.0)
    score_row = (
        jax.lax.dot_general(p2_ref[...].astype(bf), ph.astype(bf),
                            (((1,), (1,)), ((), ())),
                            preferred_element_type=f32)
        + pb2_ref[...])
    score_ref[...] = score_row.reshape(score_ref.shape)


def _make_tc_call(row_off, bc, aliased):
    nblk = bc // BLK
    off = row_off // BLK
    crow = lambda i: (i, 0)
    cdst = lambda i: (i + nblk, 0)
    grow = lambda i: (i + off, 0)
    const = lambda i: (0, 0)
    any_spec = pl.BlockSpec(memory_space=pl.ANY)
    in_specs = [
        pl.BlockSpec((BLK, MEM_DIM), crow),
        pl.BlockSpec((BLK, MEM_DIM), cdst),
        pl.BlockSpec((BLK, 1), grow),
        pl.BlockSpec((MEM_DIM, HID), const),
        pl.BlockSpec((MEM_DIM, HID), const),
        pl.BlockSpec((1, HID), const),
        pl.BlockSpec((1, HID), const),
        pl.BlockSpec((HID, MEM_DIM), const),
        pl.BlockSpec((1, MEM_DIM), const),
        pl.BlockSpec((MEM_DIM, 3 * MEM_DIM), const),
        pl.BlockSpec((1, 3 * MEM_DIM), const),
        pl.BlockSpec((MEM_DIM, 3 * MEM_DIM), const),
        pl.BlockSpec((1, 3 * MEM_DIM), const),
        pl.BlockSpec((MEM_DIM, HID), const),
        pl.BlockSpec((MEM_DIM, HID), const),
        pl.BlockSpec((1, HID), const),
        pl.BlockSpec((1, HID), const),
        pl.BlockSpec((1, 1), const),
    ]
    if aliased:
        in_specs += [any_spec, any_spec, any_spec]
    out_specs = [
        pl.BlockSpec((1, 1, BLK), lambda i: (i + off, 0, 0)),
        pl.BlockSpec((BLK, MEM_DIM), grow),
        pl.BlockSpec((BLK, MEM_DIM), grow),
    ]
    out_shape = [
        jax.ShapeDtypeStruct((B // BLK, 1, BLK), jnp.float32),
        jax.ShapeDtypeStruct((B, MEM_DIM), jnp.float32),
        jax.ShapeDtypeStruct((B, MEM_DIM), jnp.float32),
    ]
    if aliased:
        body = _tc_body
    else:
        def body(*args):
            return _tc_body(*args[:18], None, None, None, *args[18:])
    return pl.pallas_call(
        body,
        grid=(nblk,),
        in_specs=in_specs,
        out_specs=out_specs,
        out_shape=out_shape,
        input_output_aliases={18: 0, 19: 1, 20: 2} if aliased else {},
        compiler_params=pltpu.CompilerParams(
            dimension_semantics=("arbitrary",),
        ),
    )


def kernel(src, dst, time, memory, mmlp_w1, mmlp_b1, mmlp_w2, mmlp_b2,
           gru_w_ih, gru_w_hh, gru_b_ih, gru_b_hh,
           pred_w1, pred_b1, pred_w2, pred_b2):
    src = src.astype(jnp.int32)
    dst = dst.astype(jnp.int32)
    t = time.reshape(B, 1)
    w1s = mmlp_w1[:, :MEM_DIM].T
    w1d = mmlp_w1[:, MEM_DIM:2 * MEM_DIM].T
    w1t = mmlp_w1[:, 2 * MEM_DIM].reshape(1, HID)
    b1 = mmlp_b1.reshape(1, HID)
    w2 = mmlp_w2.T
    b2 = mmlp_b2.reshape(1, MEM_DIM)
    gate_scale = jnp.concatenate(
        [jnp.full((1, 2 * MEM_DIM), 0.5, jnp.float32),
         jnp.ones((1, MEM_DIM), jnp.float32)], axis=1)
    wih = gru_w_ih.T * gate_scale
    bih = gru_b_ih.reshape(1, 3 * MEM_DIM) * gate_scale
    whh = gru_w_hh.T * gate_scale
    bhh = gru_b_hh.reshape(1, 3 * MEM_DIM) * gate_scale
    p1a = pred_w1[:, :MEM_DIM].T
    p1b = pred_w1[:, MEM_DIM:].T
    pb1 = pred_b1.reshape(1, HID)
    p2 = pred_w2.reshape(1, HID)
    pb2 = pred_b2.reshape(1, 1)

    offs = [0]
    for bc in CHUNK_ROWS[:-1]:
        offs.append(offs[-1] + bc)

    staged = []
    for off, bc in zip(offs, CHUNK_ROWS):
        idx_k = jnp.concatenate(
            [lax.dynamic_slice_in_dim(src, off, bc),
             lax.dynamic_slice_in_dim(dst, off, bc)])
        staged.append(_sc_gather(memory, idx_k, 2 * bc))

    weights = (w1s, w1d, w1t, b1, w2, b2, wih, bih, whh, bhh,
               p1a, p1b, pb1, p2, pb2)
    score = ns = nd = None
    for k, (off, bc) in enumerate(zip(offs, CHUNK_ROWS)):
        if k == 0:
            score, ns, nd = _make_tc_call(off, bc, aliased=False)(
                staged[k], staged[k], t, *weights)
        else:
            score, ns, nd = _make_tc_call(off, bc, aliased=True)(
                staged[k], staged[k], t, *weights, score, ns, nd)
    return (score.reshape(B, 1), ns, nd)
